# baseline (speedup 1.0000x reference)
import functools

import jax
import jax.numpy as jnp
from jax import lax
from jax.experimental import pallas as pl
from jax.experimental.pallas import tpu as pltpu
from jax.experimental.pallas import tpu_sc as plsc

N = 4096
K = 16
NUM_RES = 200
D_H = 64
H = 4
DH = D_H // H
BINS = 100
L = 4
D_OUT = 16
OUT_CH = 2
MIN_D = 0.0
MAX_D = 10.0

BLK = 256
NBLK = N // BLK
PARTS = 4
PN = N // PARTS
F32 = jnp.float32


def _prep_body(a_ref, r_ref, e_ref, w_ref, tab_ref):
    w = w_ref[:]
    ta = jnp.dot(a_ref[:], w[0:16, :], preferred_element_type=F32)
    tr = jnp.dot(r_ref[:], w[16:24, :], preferred_element_type=F32)
    te = jnp.dot(e_ref[:], w[24:32, :], preferred_element_type=F32)
    full = (ta[:, None, None, :] + tr[None, :, None, :] + te[None, None, :, :])
    tab_ref[:] = full.reshape(85 * 4 * 10, 2 * D_H)


def _prep(atom_emb, res_emb, elem_emb, W_in):
    return pl.pallas_call(
        _prep_body,
        out_shape=jax.ShapeDtypeStruct((85 * 4 * 10, 2 * D_H), F32),
    )(atom_emb, res_emb, elem_emb, W_in)


def _sc_embed(tab, ci):
    info = plsc.get_sparse_core_info()
    nw = info.num_cores * info.num_subcores
    bpw = N // nw
    mesh = plsc.VectorSubcoreMesh(core_axis_name="c", subcore_axis_name="s")

    @functools.partial(
        pl.kernel,
        mesh=mesh,
        out_type=jax.ShapeDtypeStruct((N, 2 * D_H), F32),
        scratch_types=[
            pltpu.VMEM((bpw,), jnp.int32),
            pltpu.VMEM((bpw, 2 * D_H), F32),
            pltpu.SemaphoreType.DMA,
        ],
    )
    def run(tab_h, ci_h, out_h, idx_v, rows_v, sem):
        wid = lax.axis_index("s") * info.num_cores + lax.axis_index("c")
        base = wid * bpw
        pltpu.sync_copy(ci_h.at[pl.ds(base, bpw)], idx_v)
        pltpu.async_copy(tab_h.at[idx_v], rows_v, sem).wait()
        pltpu.sync_copy(rows_v, out_h.at[pl.ds(base, bpw)])

    return run(tab, ci)


def _sc_gather_kv(kv, idx_flat):
    info = plsc.get_sparse_core_info()
    nw = info.num_cores * info.num_subcores
    nrows = idx_flat.shape[0]
    bpw = nrows // nw
    chunk = min(bpw, 512)
    nch = bpw // chunk
    mesh = plsc.VectorSubcoreMesh(core_axis_name="c", subcore_axis_name="s")

    @functools.partial(
        pl.kernel,
        mesh=mesh,
        out_type=jax.ShapeDtypeStruct((nrows, 2 * D_H), F32),
        scratch_types=[
            pltpu.VMEM((chunk,), jnp.int32),
            pltpu.VMEM((chunk, 2 * D_H), F32),
            pltpu.SemaphoreType.DMA,
        ],
    )
    def run(kv_h, idx_h, out_h, idx_v, rows_v, sem):
        wid = lax.axis_index("s") * info.num_cores + lax.axis_index("c")
        base = wid * bpw
        for c in range(nch):
            off = base + c * chunk
            pltpu.sync_copy(idx_h.at[pl.ds(off, chunk)], idx_v)
            pltpu.async_copy(kv_h.at[idx_v], rows_v, sem).wait()
            pltpu.sync_copy(rows_v, out_h.at[pl.ds(off, chunk)])

    return run(kv, idx_flat)


def _knn_body(half, cpt_ref, cp_ref, b2_ref, idx_ref, b0_ref, b1_ref,
              b2o_ref, b3_ref):
    bias_refs = (b0_ref, b1_ref, b2o_ref, b3_ref)
    cpb = cp_ref[:]
    ct = cpt_ref[:]
    x2b = jnp.sum(cpb * cpb, axis=1, keepdims=True)
    x2a = jnp.sum(ct * ct, axis=0, keepdims=True)
    cross = jax.lax.dot_general(cpb, ct, (((1,), (0,)), ((), ())),
                                preferred_element_type=F32)
    d2 = x2b + x2a - 2.0 * cross
    cols = lax.broadcasted_iota(jnp.int32, (KBLK, N), 1)
    rows = (lax.broadcasted_iota(jnp.int32, (KBLK, 1), 0)
            + pl.program_id(0) * KBLK + half * PN)
    inf = jnp.float32(jnp.inf)
    colsf = cols.astype(F32)
    work = jnp.where(cols == rows, inf, d2)

    step = jnp.float32((MAX_D - MIN_D) / (BINS - 1))
    centers = lax.broadcasted_iota(jnp.int32, (1, BINS), 1).astype(F32) * step
    sigma = (MAX_D - MIN_D) / BINS
    inv2s2 = jnp.float32(1.0 / (2.0 * sigma * sigma))
    b2 = b2_ref[:]

    idx_cols = []
    for t in range(K):
        m = jnp.min(work, axis=1, keepdims=True)
        eq = work == m
        colf = jnp.min(jnp.where(eq, colsf, jnp.float32(N)),
                       axis=1, keepdims=True)
        work = jnp.where(eq, inf, work)
        idx_cols.append(colf.astype(jnp.int32))
        d_t = jnp.sqrt(jnp.maximum(m, 1e-12))
        rbf_t = jnp.exp(-((d_t - centers) ** 2) * inv2s2)
        bias_t = jax.lax.dot_general(rbf_t, b2, (((1,), (0,)), ((), ())),
                                     preferred_element_type=F32)
        for l in range(L):
            bias_refs[l][:, t * H:t * H + H] = bias_t[:, l * H:(l + 1) * H]
    idx_ref[:] = jnp.concatenate(idx_cols, axis=1)


KBLK = 256


def _knn(cpT, cp, B2, half):
    hn = PN
    nblk = hn // KBLK
    return pl.pallas_call(
        functools.partial(_knn_body, half),
        grid=(nblk,),
        in_specs=[
            pl.BlockSpec((8, N), lambda i: (0, 0)),
            pl.BlockSpec((KBLK, 8), lambda i, _h=half: (i + _h * nblk, 0)),
            pl.BlockSpec((BINS, L * H), lambda i: (0, 0)),
        ],
        out_specs=[pl.BlockSpec((KBLK, K), lambda i: (i, 0))]
        + [pl.BlockSpec((KBLK, K * H), lambda i: (i, 0)) for _ in range(L)],
        out_shape=[jax.ShapeDtypeStruct((hn, K), jnp.int32)]
        + [jax.ShapeDtypeStruct((hn, K * H), F32) for _ in range(L)],
    )(cpT, cp, B2)


def _qkv0_body(g_ref, wq_ref, wkv_ref, h_ref, q_ref, kv_ref):
    h0 = g_ref[:, :D_H]
    h_ref[:] = h0
    q_ref[:] = jnp.dot(h0, wq_ref[:], preferred_element_type=F32)
    kv_ref[:] = jnp.dot(h0, wkv_ref[:], preferred_element_type=F32)


def _qkv0(g, wq, wkv, half):
    hn = PN
    nblk = hn // BLK
    return pl.pallas_call(
        _qkv0_body,
        grid=(nblk,),
        in_specs=[
            pl.BlockSpec((BLK, 2 * D_H), lambda i, _h=half: (i + _h * nblk, 0)),
            pl.BlockSpec((D_H, D_H), lambda i: (0, 0)),
            pl.BlockSpec((D_H, 2 * D_H), lambda i: (0, 0)),
        ],
        out_specs=[
            pl.BlockSpec((BLK, D_H), lambda i: (i, 0)),
            pl.BlockSpec((BLK, D_H), lambda i: (i, 0)),
            pl.BlockSpec((BLK, 2 * D_H), lambda i: (i, 0)),
        ],
        out_shape=[
            jax.ShapeDtypeStruct((hn, D_H), F32),
            jax.ShapeDtypeStruct((hn, D_H), F32),
            jax.ShapeDtypeStruct((hn, 2 * D_H), F32),
        ],
    )(g, wq, wkv)


def _attn_body(h_ref, q_ref, kvt_ref, bias_ref, wo_ref, w1_ref, w2_ref,
               wqn_ref, wkvn_ref, hn_ref, qn_ref, kvn_ref):
    q = q_ref[:]
    ks = jnp.concatenate([kvt_ref[s][:, :D_H] for s in range(K)], axis=1)
    vs = jnp.concatenate([kvt_ref[s][:, D_H:] for s in range(K)], axis=1)
    qrep = jnp.concatenate([q] * K, axis=1)
    qk = qrep * ks

    di = lax.broadcasted_iota(jnp.int32, (K * D_H, K * H), 0)
    ci = lax.broadcasted_iota(jnp.int32, (K * D_H, K * H), 1)
    gbig = jnp.where((di // D_H == ci // H) & ((di % D_H) // DH == ci % H),
                     1.0, 0.0).astype(F32)
    logits = (jnp.dot(qk, gbig, preferred_element_type=F32)
              * jnp.float32(1.0 / (DH ** 0.5)) + bias_ref[:])
    ex = jnp.exp(logits)
    dc = lax.broadcasted_iota(jnp.int32, (K * H, K * H), 0)
    dcc = lax.broadcasted_iota(jnp.int32, (K * H, K * H), 1)
    dmat = jnp.where(dc % H == dcc % H, 1.0, 0.0).astype(F32)
    den = jnp.dot(ex, dmat, preferred_element_type=F32)
    alpha = ex / (den + 1e-9)
    ec = lax.broadcasted_iota(jnp.int32, (K * H, K * D_H), 0)
    ed = lax.broadcasted_iota(jnp.int32, (K * H, K * D_H), 1)
    ebig = jnp.where((ec // H == ed // D_H) & (ec % H == (ed % D_H) // DH),
                     1.0, 0.0).astype(F32)
    aexp = jnp.dot(alpha, ebig, preferred_element_type=F32)
    aggs = aexp * vs
    ri = lax.broadcasted_iota(jnp.int32, (K * D_H, D_H), 0)
    rf = lax.broadcasted_iota(jnp.int32, (K * D_H, D_H), 1)
    rmat = jnp.where(ri % D_H == rf, 1.0, 0.0).astype(F32)
    agg = jnp.dot(aggs, rmat, preferred_element_type=F32)

    h1 = h_ref[:] + jnp.dot(agg, wo_ref[:], preferred_element_type=F32)
    t = jnp.maximum(jnp.dot(h1, w1_ref[:], preferred_element_type=F32), 0.0)
    h2 = h1 + jnp.dot(t, w2_ref[:], preferred_element_type=F32)
    hn_ref[:] = h2
    qn_ref[:] = jnp.dot(h2, wqn_ref[:], preferred_element_type=F32)
    kvn_ref[:] = jnp.dot(h2, wkvn_ref[:], preferred_element_type=F32)


def _attn(h, q, kvt, bias, wo, w1, w2, wqn, wkvn):
    hn = h.shape[0]
    return pl.pallas_call(
        _attn_body,
        grid=(hn // BLK,),
        in_specs=[
            pl.BlockSpec((BLK, D_H), lambda i: (i, 0)),
            pl.BlockSpec((BLK, D_H), lambda i: (i, 0)),
            pl.BlockSpec((K, BLK, 2 * D_H), lambda i: (0, i, 0)),
            pl.BlockSpec((BLK, K * H), lambda i: (i, 0)),
            pl.BlockSpec((D_H, D_H), lambda i: (0, 0)),
            pl.BlockSpec((D_H, 2 * D_H), lambda i: (0, 0)),
            pl.BlockSpec((2 * D_H, D_H), lambda i: (0, 0)),
            pl.BlockSpec((D_H, D_H), lambda i: (0, 0)),
            pl.BlockSpec((D_H, 2 * D_H), lambda i: (0, 0)),
        ],
        out_specs=[
            pl.BlockSpec((BLK, D_H), lambda i: (i, 0)),
            pl.BlockSpec((BLK, D_H), lambda i: (i, 0)),
            pl.BlockSpec((BLK, 2 * D_H), lambda i: (i, 0)),
        ],
        out_shape=[
            jax.ShapeDtypeStruct((hn, D_H), F32),
            jax.ShapeDtypeStruct((hn, D_H), F32),
            jax.ShapeDtypeStruct((hn, 2 * D_H), F32),
        ],
    )(h, q, kvt, bias, wo, w1, w2, wqn, wkvn)


def _final_body(ha_ref, hb_ref, hc_ref, hd_ref, rid_ref, wrep_ref,
                wproj_ref, bproj_ref, out_ref):
    h = jnp.concatenate([ha_ref[:], hb_ref[:], hc_ref[:], hd_ref[:]], axis=0)
    out_atoms = jnp.dot(h, wrep_ref[:], preferred_element_type=F32)
    rid = rid_ref[:]
    seg = lax.broadcasted_iota(jnp.int32, (N, NUM_RES), 1)
    onehot = jnp.where(seg == rid, 1.0, 0.0).astype(F32)
    counts = jnp.sum(onehot, axis=0, keepdims=True)
    segsum = jax.lax.dot_general(onehot, out_atoms, (((0,), (0,)), ((), ())),
                                 preferred_element_type=F32)
    res_feat = segsum / jnp.maximum(counts, 1.0).reshape(NUM_RES, 1)
    out_ref[:] = (jnp.dot(res_feat, wproj_ref[:], preferred_element_type=F32)
                  + bproj_ref[:])


def _final(hs, rid, W_outrep, W_proj, b_proj):
    return pl.pallas_call(
        _final_body,
        out_shape=jax.ShapeDtypeStruct((NUM_RES, OUT_CH), F32),
    )(*hs, rid, W_outrep, W_proj, b_proj)


def kernel(coords, atom_idx, element_idx, residue_type, residue_ids, atom_emb,
           elem_emb, res_emb, W_in, Wq, Wk, Wv, Wo, B_bias, W1, W2, W_outrep,
           W_proj, b_proj):
    cp = jnp.pad(coords.astype(F32), ((0, 0), (0, 5)))
    cpT = cp.T
    B2 = jnp.transpose(B_bias.astype(F32), (1, 0, 2)).reshape(BINS, L * H)
    wkv = [jnp.concatenate([Wk[l], Wv[l]], axis=1).astype(F32)
           for l in range(L)]

    w_in_pad = jnp.pad(W_in.astype(F32), ((0, 0), (0, D_H)))
    tab = _prep(atom_emb.astype(F32), res_emb.astype(F32),
                elem_emb.astype(F32), w_in_pad)
    ci = (atom_idx.astype(jnp.int32) * 40
          + residue_type.astype(jnp.int32) * 10
          + element_idx.astype(jnp.int32))
    g = _sc_embed(tab, ci)
    idxs, biases, iflats = [], [], []
    for p in range(PARTS):
        idxp, *biasp = _knn(cpT, cp, B2, p)
        idxs.append(idxp)
        biases.append(biasp)
        iflats.append(idxp.T.reshape(PN * K))

    hs, qs, kvs = [], [], []
    for p in range(PARTS):
        hp, qp, kvp = _qkv0(g, Wq[0].astype(F32), wkv[0], p)
        hs.append(hp)
        qs.append(qp)
        kvs.append(kvp)
    kv = jnp.concatenate(kvs, axis=0)
    for l in range(L):
        wo = Wo[l].astype(F32)
        w1 = W1[l].astype(F32)
        w2 = W2[l].astype(F32)
        wqn = Wq[(l + 1) % L].astype(F32)
        wkvn = wkv[(l + 1) % L]
        es = [_sc_gather_kv(kv, iflats[p]) for p in range(PARTS)]
        for p in range(PARTS):
            hs[p], qs[p], kvs[p] = _attn(
                hs[p], qs[p], es[p].reshape(K, PN, 2 * D_H), biases[p][l],
                wo, w1, w2, wqn, wkvn)
        kv = jnp.concatenate(kvs, axis=0)

    out = _final(hs, residue_ids.astype(jnp.int32).reshape(N, 1),
                 W_outrep.astype(F32), W_proj.astype(F32),
                 b_proj.astype(F32).reshape(1, OUT_CH))
    return out


# speedup vs baseline: 1.0351x; 1.0351x over previous
import functools

import jax
import jax.numpy as jnp
from jax import lax
from jax.experimental import pallas as pl
from jax.experimental.pallas import tpu as pltpu
from jax.experimental.pallas import tpu_sc as plsc

N = 4096
K = 16
NUM_RES = 200
D_H = 64
H = 4
DH = D_H // H
BINS = 100
L = 4
D_OUT = 16
OUT_CH = 2
MIN_D = 0.0
MAX_D = 10.0

BLK = 256
NBLK = N // BLK
PARTS = 2
PN = N // PARTS
F32 = jnp.float32


def _prep_body(a_ref, r_ref, e_ref, w_ref, tab_ref):
    w = w_ref[:]
    ta = jnp.dot(a_ref[:], w[0:16, :], preferred_element_type=F32)
    tr = jnp.dot(r_ref[:], w[16:24, :], preferred_element_type=F32)
    te = jnp.dot(e_ref[:], w[24:32, :], preferred_element_type=F32)
    full = (ta[:, None, None, :] + tr[None, :, None, :] + te[None, None, :, :])
    tab_ref[:] = full.reshape(85 * 4 * 10, 2 * D_H)


def _prep(atom_emb, res_emb, elem_emb, W_in):
    return pl.pallas_call(
        _prep_body,
        out_shape=jax.ShapeDtypeStruct((85 * 4 * 10, 2 * D_H), F32),
    )(atom_emb, res_emb, elem_emb, W_in)


def _sc_embed(tab, ci):
    info = plsc.get_sparse_core_info()
    nw = info.num_cores * info.num_subcores
    bpw = N // nw
    mesh = plsc.VectorSubcoreMesh(core_axis_name="c", subcore_axis_name="s")

    @functools.partial(
        pl.kernel,
        mesh=mesh,
        out_type=jax.ShapeDtypeStruct((N, 2 * D_H), F32),
        scratch_types=[
            pltpu.VMEM((bpw,), jnp.int32),
            pltpu.VMEM((bpw, 2 * D_H), F32),
            pltpu.SemaphoreType.DMA,
        ],
    )
    def run(tab_h, ci_h, out_h, idx_v, rows_v, sem):
        wid = lax.axis_index("s") * info.num_cores + lax.axis_index("c")
        base = wid * bpw
        pltpu.sync_copy(ci_h.at[pl.ds(base, bpw)], idx_v)
        pltpu.async_copy(tab_h.at[idx_v], rows_v, sem).wait()
        pltpu.sync_copy(rows_v, out_h.at[pl.ds(base, bpw)])

    return run(tab, ci)


def _sc_gather_kv(kv, idx_flat):
    info = plsc.get_sparse_core_info()
    nw = info.num_cores * info.num_subcores
    nrows = idx_flat.shape[0]
    bpw = nrows // nw
    chunk = min(bpw, 512)
    nch = bpw // chunk
    mesh = plsc.VectorSubcoreMesh(core_axis_name="c", subcore_axis_name="s")

    @functools.partial(
        pl.kernel,
        mesh=mesh,
        out_type=jax.ShapeDtypeStruct((nrows, 2 * D_H), F32),
        scratch_types=[
            pltpu.VMEM((chunk,), jnp.int32),
            pltpu.VMEM((chunk, 2 * D_H), F32),
            pltpu.SemaphoreType.DMA,
        ],
    )
    def run(kv_h, idx_h, out_h, idx_v, rows_v, sem):
        wid = lax.axis_index("s") * info.num_cores + lax.axis_index("c")
        base = wid * bpw
        for c in range(nch):
            off = base + c * chunk
            pltpu.sync_copy(idx_h.at[pl.ds(off, chunk)], idx_v)
            pltpu.async_copy(kv_h.at[idx_v], rows_v, sem).wait()
            pltpu.sync_copy(rows_v, out_h.at[pl.ds(off, chunk)])

    return run(kv, idx_flat)


def _knn_body(half, cpt_ref, cp_ref, b2_ref, idx_ref, b0_ref, b1_ref,
              b2o_ref, b3_ref):
    bias_refs = (b0_ref, b1_ref, b2o_ref, b3_ref)
    cpb = cp_ref[:]
    ct = cpt_ref[:]
    x2b = jnp.sum(cpb * cpb, axis=1, keepdims=True)
    x2a = jnp.sum(ct * ct, axis=0, keepdims=True)
    cross = jax.lax.dot_general(cpb, ct, (((1,), (0,)), ((), ())),
                                preferred_element_type=F32)
    d2 = x2b + x2a - 2.0 * cross
    cols = lax.broadcasted_iota(jnp.int32, (KBLK, N), 1)
    rows = (lax.broadcasted_iota(jnp.int32, (KBLK, 1), 0)
            + pl.program_id(0) * KBLK + half * PN)
    inf = jnp.float32(jnp.inf)
    colsf = cols.astype(F32)
    work = jnp.where(cols == rows, inf, d2)

    step = jnp.float32((MAX_D - MIN_D) / (BINS - 1))
    centers = lax.broadcasted_iota(jnp.int32, (1, BINS), 1).astype(F32) * step
    sigma = (MAX_D - MIN_D) / BINS
    inv2s2 = jnp.float32(1.0 / (2.0 * sigma * sigma))
    b2 = b2_ref[:]

    idx_cols = []
    for t in range(K):
        m = jnp.min(work, axis=1, keepdims=True)
        eq = work == m
        colf = jnp.min(jnp.where(eq, colsf, jnp.float32(N)),
                       axis=1, keepdims=True)
        work = jnp.where(eq, inf, work)
        idx_cols.append(colf.astype(jnp.int32))
        d_t = jnp.sqrt(jnp.maximum(m, 1e-12))
        rbf_t = jnp.exp(-((d_t - centers) ** 2) * inv2s2)
        bias_t = jax.lax.dot_general(rbf_t, b2, (((1,), (0,)), ((), ())),
                                     preferred_element_type=F32)
        for l in range(L):
            bias_refs[l][:, t * H:t * H + H] = bias_t[:, l * H:(l + 1) * H]
    idx_ref[:] = jnp.concatenate(idx_cols, axis=1)


KBLK = 256


def _knn(cpT, cp, B2, half):
    hn = PN
    nblk = hn // KBLK
    return pl.pallas_call(
        functools.partial(_knn_body, half),
        grid=(nblk,),
        in_specs=[
            pl.BlockSpec((8, N), lambda i: (0, 0)),
            pl.BlockSpec((KBLK, 8), lambda i, _h=half: (i + _h * nblk, 0)),
            pl.BlockSpec((BINS, L * H), lambda i: (0, 0)),
        ],
        out_specs=[pl.BlockSpec((KBLK, K), lambda i: (i, 0))]
        + [pl.BlockSpec((KBLK, K * H), lambda i: (i, 0)) for _ in range(L)],
        out_shape=[jax.ShapeDtypeStruct((hn, K), jnp.int32)]
        + [jax.ShapeDtypeStruct((hn, K * H), F32) for _ in range(L)],
    )(cpT, cp, B2)


def _qkv0_body(g_ref, wq_ref, wkv_ref, h_ref, q_ref, kv_ref):
    h0 = g_ref[:, :D_H]
    h_ref[:] = h0
    q_ref[:] = jnp.dot(h0, wq_ref[:], preferred_element_type=F32)
    kv_ref[:] = jnp.dot(h0, wkv_ref[:], preferred_element_type=F32)


def _qkv0(g, wq, wkv, half):
    hn = PN
    nblk = hn // BLK
    return pl.pallas_call(
        _qkv0_body,
        grid=(nblk,),
        in_specs=[
            pl.BlockSpec((BLK, 2 * D_H), lambda i, _h=half: (i + _h * nblk, 0)),
            pl.BlockSpec((D_H, D_H), lambda i: (0, 0)),
            pl.BlockSpec((D_H, 2 * D_H), lambda i: (0, 0)),
        ],
        out_specs=[
            pl.BlockSpec((BLK, D_H), lambda i: (i, 0)),
            pl.BlockSpec((BLK, D_H), lambda i: (i, 0)),
            pl.BlockSpec((BLK, 2 * D_H), lambda i: (i, 0)),
        ],
        out_shape=[
            jax.ShapeDtypeStruct((hn, D_H), F32),
            jax.ShapeDtypeStruct((hn, D_H), F32),
            jax.ShapeDtypeStruct((hn, 2 * D_H), F32),
        ],
    )(g, wq, wkv)


def _attn_body(h_ref, q_ref, kvt_ref, bias_ref, wo_ref, w1_ref, w2_ref,
               wqn_ref, wkvn_ref, hn_ref, qn_ref, kvn_ref):
    q = q_ref[:]
    ks = jnp.concatenate([kvt_ref[s][:, :D_H] for s in range(K)], axis=1)
    vs = jnp.concatenate([kvt_ref[s][:, D_H:] for s in range(K)], axis=1)
    qrep = jnp.concatenate([q] * K, axis=1)
    qk = qrep * ks

    di = lax.broadcasted_iota(jnp.int32, (K * D_H, K * H), 0)
    ci = lax.broadcasted_iota(jnp.int32, (K * D_H, K * H), 1)
    gbig = jnp.where((di // D_H == ci // H) & ((di % D_H) // DH == ci % H),
                     1.0, 0.0).astype(F32)
    logits = (jnp.dot(qk, gbig, preferred_element_type=F32)
              * jnp.float32(1.0 / (DH ** 0.5)) + bias_ref[:])
    ex = jnp.exp(logits)
    dc = lax.broadcasted_iota(jnp.int32, (K * H, K * H), 0)
    dcc = lax.broadcasted_iota(jnp.int32, (K * H, K * H), 1)
    dmat = jnp.where(dc % H == dcc % H, 1.0, 0.0).astype(F32)
    den = jnp.dot(ex, dmat, preferred_element_type=F32)
    alpha = ex / (den + 1e-9)
    ec = lax.broadcasted_iota(jnp.int32, (K * H, K * D_H), 0)
    ed = lax.broadcasted_iota(jnp.int32, (K * H, K * D_H), 1)
    ebig = jnp.where((ec // H == ed // D_H) & (ec % H == (ed % D_H) // DH),
                     1.0, 0.0).astype(F32)
    aexp = jnp.dot(alpha, ebig, preferred_element_type=F32)
    aggs = aexp * vs
    ri = lax.broadcasted_iota(jnp.int32, (K * D_H, D_H), 0)
    rf = lax.broadcasted_iota(jnp.int32, (K * D_H, D_H), 1)
    rmat = jnp.where(ri % D_H == rf, 1.0, 0.0).astype(F32)
    agg = jnp.dot(aggs, rmat, preferred_element_type=F32)

    h1 = h_ref[:] + jnp.dot(agg, wo_ref[:], preferred_element_type=F32)
    t = jnp.maximum(jnp.dot(h1, w1_ref[:], preferred_element_type=F32), 0.0)
    h2 = h1 + jnp.dot(t, w2_ref[:], preferred_element_type=F32)
    hn_ref[:] = h2
    qn_ref[:] = jnp.dot(h2, wqn_ref[:], preferred_element_type=F32)
    kvn_ref[:] = jnp.dot(h2, wkvn_ref[:], preferred_element_type=F32)


def _attn(h, q, kvt, bias, wo, w1, w2, wqn, wkvn):
    hn = h.shape[0]
    return pl.pallas_call(
        _attn_body,
        grid=(hn // BLK,),
        in_specs=[
            pl.BlockSpec((BLK, D_H), lambda i: (i, 0)),
            pl.BlockSpec((BLK, D_H), lambda i: (i, 0)),
            pl.BlockSpec((K, BLK, 2 * D_H), lambda i: (0, i, 0)),
            pl.BlockSpec((BLK, K * H), lambda i: (i, 0)),
            pl.BlockSpec((D_H, D_H), lambda i: (0, 0)),
            pl.BlockSpec((D_H, 2 * D_H), lambda i: (0, 0)),
            pl.BlockSpec((2 * D_H, D_H), lambda i: (0, 0)),
            pl.BlockSpec((D_H, D_H), lambda i: (0, 0)),
            pl.BlockSpec((D_H, 2 * D_H), lambda i: (0, 0)),
        ],
        out_specs=[
            pl.BlockSpec((BLK, D_H), lambda i: (i, 0)),
            pl.BlockSpec((BLK, D_H), lambda i: (i, 0)),
            pl.BlockSpec((BLK, 2 * D_H), lambda i: (i, 0)),
        ],
        out_shape=[
            jax.ShapeDtypeStruct((hn, D_H), F32),
            jax.ShapeDtypeStruct((hn, D_H), F32),
            jax.ShapeDtypeStruct((hn, 2 * D_H), F32),
        ],
    )(h, q, kvt, bias, wo, w1, w2, wqn, wkvn)


def _final_body(ha_ref, hb_ref, rid_ref, wrep_ref, wproj_ref, bproj_ref,
                out_ref):
    h = jnp.concatenate([ha_ref[:], hb_ref[:]], axis=0)
    out_atoms = jnp.dot(h, wrep_ref[:], preferred_element_type=F32)
    rid = rid_ref[:]
    seg = lax.broadcasted_iota(jnp.int32, (N, NUM_RES), 1)
    onehot = jnp.where(seg == rid, 1.0, 0.0).astype(F32)
    counts = jnp.sum(onehot, axis=0, keepdims=True)
    segsum = jax.lax.dot_general(onehot, out_atoms, (((0,), (0,)), ((), ())),
                                 preferred_element_type=F32)
    res_feat = segsum / jnp.maximum(counts, 1.0).reshape(NUM_RES, 1)
    out_ref[:] = (jnp.dot(res_feat, wproj_ref[:], preferred_element_type=F32)
                  + bproj_ref[:])


def _final(hs, rid, W_outrep, W_proj, b_proj):
    return pl.pallas_call(
        _final_body,
        out_shape=jax.ShapeDtypeStruct((NUM_RES, OUT_CH), F32),
    )(*hs, rid, W_outrep, W_proj, b_proj)


def kernel(coords, atom_idx, element_idx, residue_type, residue_ids, atom_emb,
           elem_emb, res_emb, W_in, Wq, Wk, Wv, Wo, B_bias, W1, W2, W_outrep,
           W_proj, b_proj):
    cp = jnp.pad(coords.astype(F32), ((0, 0), (0, 5)))
    cpT = cp.T
    B2 = jnp.transpose(B_bias.astype(F32), (1, 0, 2)).reshape(BINS, L * H)
    wkv = [jnp.concatenate([Wk[l], Wv[l]], axis=1).astype(F32)
           for l in range(L)]

    w_in_pad = jnp.pad(W_in.astype(F32), ((0, 0), (0, D_H)))
    tab = _prep(atom_emb.astype(F32), res_emb.astype(F32),
                elem_emb.astype(F32), w_in_pad)
    ci = (atom_idx.astype(jnp.int32) * 40
          + residue_type.astype(jnp.int32) * 10
          + element_idx.astype(jnp.int32))
    g = _sc_embed(tab, ci)
    idxs, biases, iflats = [], [], []
    for p in range(PARTS):
        idxp, *biasp = _knn(cpT, cp, B2, p)
        idxs.append(idxp)
        biases.append(biasp)
        iflats.append(idxp.T.reshape(PN * K))

    hs, qs, kvs = [], [], []
    for p in range(PARTS):
        hp, qp, kvp = _qkv0(g, Wq[0].astype(F32), wkv[0], p)
        hs.append(hp)
        qs.append(qp)
        kvs.append(kvp)
    kv = jnp.concatenate(kvs, axis=0)
    for l in range(L):
        wo = Wo[l].astype(F32)
        w1 = W1[l].astype(F32)
        w2 = W2[l].astype(F32)
        wqn = Wq[(l + 1) % L].astype(F32)
        wkvn = wkv[(l + 1) % L]
        es = [_sc_gather_kv(kv, iflats[p]) for p in range(PARTS)]
        for p in range(PARTS):
            hs[p], qs[p], kvs[p] = _attn(
                hs[p], qs[p], es[p].reshape(K, PN, 2 * D_H), biases[p][l],
                wo, w1, w2, wqn, wkvn)
        kv = jnp.concatenate(kvs, axis=0)

    out = _final(hs, residue_ids.astype(jnp.int32).reshape(N, 1),
                 W_outrep.astype(F32), W_proj.astype(F32),
                 b_proj.astype(F32).reshape(1, OUT_CH))
    return out
